# baseline (speedup 1.0000x reference)
import functools

import numpy as np
import jax
import jax.numpy as jnp
from jax import lax
from jax.experimental import pallas as pl
from jax.experimental.pallas import tpu as pltpu
from jax.experimental.pallas import tpu_sc as plsc

_B = 4096
_NT = 26
_V = 100000
_D = 16
_NE = _NT + 1
_NEP = 32
_ZW = _NEP * _NEP
_BB = 512
_GRID = _B // _BB

_NC, _NS = 2, 16
_NW = _NC * _NS
_CH = 128
_NSPLIT = 2
_NTH = _NT // _NSPLIT
_EPS = _B * _NTH * _D
_CPW = _EPS // _NW // _CH
_GRP = 16


def _sc_gather(tbl1d, eidx3):
    mesh = plsc.VectorSubcoreMesh(core_axis_name="c", subcore_axis_name="s")

    @functools.partial(
        pl.kernel,
        mesh=mesh,
        out_type=jax.ShapeDtypeStruct((_NW, _CPW, _CH), jnp.float32),
        scratch_types=[
            pltpu.VMEM((_CPW, _CH), jnp.int32),
            pltpu.VMEM((_CPW, _CH), jnp.float32),
            pltpu.SemaphoreType.DMA,
        ],
        compiler_params=pltpu.CompilerParams(use_tc_tiling_on_sc=False),
    )
    def k(tbl_hbm, idx_hbm, out_hbm, idx_v, out_v, sem):
        wid = lax.axis_index("s") * _NC + lax.axis_index("c")
        pltpu.sync_copy(idx_hbm.at[wid], idx_v)

        def grp(c, carry):
            copies = [
                pltpu.async_copy(
                    tbl_hbm.at[idx_v.at[c * _GRP + k_]],
                    out_v.at[c * _GRP + k_], sem)
                for k_ in range(_GRP)
            ]
            for cp in copies:
                cp.wait()
            return carry

        lax.fori_loop(0, _CPW // _GRP, grp, 0)
        pltpu.sync_copy(out_v, out_hbm.at[wid])

    return k(tbl1d, eidx3)


def _dot(a, b, dn):
    return lax.dot_general(a, b, (dn, ((), ())),
                           preferred_element_type=jnp.float32)


def _tc_body(dx, emb, tgt, wb1, bb1, wb2, bb2, wb3, bb3, wb4, bb4,
             mx, me, wz2, wx, bt1, wt2, bt2, wt3, bt3, out):
    x = dx[...]
    x = jnp.maximum(_dot(x, wb1[...], ((1,), (1,))) + bb1[...], 0.0)
    x = jnp.maximum(_dot(x, wb2[...], ((1,), (1,))) + bb2[...], 0.0)
    x = jnp.maximum(_dot(x, wb3[...], ((1,), (1,))) + bb3[...], 0.0)
    x4 = jnp.maximum(_dot(x, wb4[...], ((1,), (1,))) + bb4[...], 0.0)

    e = emb[...]
    tt = _dot(mx[...], x4, ((1,), (1,))) + _dot(me[...], e, ((1,), (1,)))
    a = tt.reshape(_D, _NEP, _BB)

    z = None
    for d in range(_D):
        ad = a[d]
        term = ad[:, None, :] * ad[None, :, :]
        z = term if z is None else z + term
    zt = z.reshape(_ZW, _BB)

    h = _dot(wx[...], x4, ((1,), (1,))) + _dot(wz2[...], zt, ((1,), (0,)))
    h = jnp.maximum(h + bt1[...], 0.0)
    h = jnp.maximum(_dot(wt2[...], h, ((1,), (0,))) + bt2[...], 0.0)
    logit = _dot(wt3[...], h, ((1,), (0,))) + bt3[...]
    p = jax.nn.sigmoid(logit)

    t = tgt[0]
    eps = 1e-12
    c = (t * jnp.log(jnp.clip(p, eps, 1.0))
         + (1.0 - t) * jnp.log(jnp.clip(1.0 - p, eps, 1.0)))
    s = jnp.sum(c, axis=1, keepdims=True) * (-1.0 / _B)

    @pl.when(pl.program_id(0) == 0)
    def _():
        out[...] = jnp.zeros_like(s)

    out[...] += s


def _tc_loss(dxB, embB, tgt3, weights, interpret=False):
    full = lambda arr: pl.BlockSpec(arr.shape, lambda i: (0,) * arr.ndim)
    in_specs = [
        pl.BlockSpec((_BB, 13), lambda i: (i, 0)),
        pl.BlockSpec((_BB, _NT * _D), lambda i: (i, 0)),
        pl.BlockSpec((1, 1, _BB), lambda i: (i, 0, 0)),
    ] + [full(w) for w in weights]
    return pl.pallas_call(
        _tc_body,
        grid=(_GRID,),
        in_specs=in_specs,
        out_specs=pl.BlockSpec((1, 1), lambda i: (0, 0)),
        out_shape=jax.ShapeDtypeStruct((1, 1), jnp.float32),
        interpret=interpret,
    )(dxB, embB, tgt3, *weights)


def _selection_mats():
    mx = np.zeros((_D * _NEP, _D), np.float32)
    d = np.arange(_D)
    mx[d * _NEP, d] = 1.0
    me = np.zeros((_D * _NEP, _NT * _D), np.float32)
    ent, dd = np.meshgrid(np.arange(1, _NE), d, indexing="ij")
    me[dd.ravel() * _NEP + ent.ravel(), (ent.ravel() - 1) * _D + dd.ravel()] = 1.0
    return jnp.asarray(mx), jnp.asarray(me)


def kernel(dense_x, lS_i, target, emb_tables, bot_Ws, bot_bs, top_Ws, top_bs):
    doff = (jnp.arange(_D, dtype=jnp.int32) * _V)[None, None, :]
    toff = (jnp.arange(_NTH, dtype=jnp.int32) * (_D * _V))[None, :, None]
    embs = []
    for h in range(_NSPLIT):
        tbl1d = emb_tables[h * _NTH:(h + 1) * _NTH].transpose(0, 2, 1).reshape(-1)
        vv = lS_i[h * _NTH:(h + 1) * _NTH].T[:, :, None]
        eidx3 = (vv + toff + doff).reshape(_NW, _CPW, _CH)
        embs.append(_sc_gather(tbl1d, eidx3).reshape(_B, _NTH * _D))
    emb = jnp.concatenate(embs, axis=1)

    tgt3 = target.reshape(_GRID, 1, _BB)
    mx, me = _selection_mats()
    cols = np.array([i * _NEP + j for i in range(_NE) for j in range(i)])
    wz2 = jnp.zeros((512, _ZW), jnp.float32).at[:, cols].set(top_Ws[0][:, _D:])
    wx = top_Ws[0][:, :_D]
    weights = [
        bot_Ws[0], bot_bs[0][None, :], bot_Ws[1], bot_bs[1][None, :],
        bot_Ws[2], bot_bs[2][None, :], bot_Ws[3], bot_bs[3][None, :],
        mx, me, wz2, wx, top_bs[0][:, None],
        top_Ws[1], top_bs[1][:, None], top_Ws[2], top_bs[2][:, None],
    ]
    loss = _tc_loss(dense_x, emb, tgt3, weights)
    return loss[0, 0]


# speedup vs baseline: 1.1435x; 1.1435x over previous
import functools

import numpy as np
import jax
import jax.numpy as jnp
from jax import lax
from jax.experimental import pallas as pl
from jax.experimental.pallas import tpu as pltpu
from jax.experimental.pallas import tpu_sc as plsc

_B = 4096
_NT = 26
_V = 100000
_D = 16
_NE = _NT + 1
_NEP = 32
_ZW = _NEP * _NEP
_BB = 512
_GRID = _B // _BB

_NC, _NS = 2, 16
_NW = _NC * _NS
_CH = 128
_NSPLIT = 2
_NTH = _NT // _NSPLIT
_EPS = _B * _NTH * _D
_CPW = _EPS // _NW // _CH
_GRP = 16


def _sc_gather(tbl1d, eidx3):
    mesh = plsc.VectorSubcoreMesh(core_axis_name="c", subcore_axis_name="s")

    @functools.partial(
        pl.kernel,
        mesh=mesh,
        out_type=jax.ShapeDtypeStruct((_NW, _CPW, _CH), jnp.float32),
        scratch_types=[
            pltpu.VMEM((_CPW, _CH), jnp.int32),
            pltpu.VMEM((_CPW, _CH), jnp.float32),
            pltpu.SemaphoreType.DMA,
        ],
        compiler_params=pltpu.CompilerParams(use_tc_tiling_on_sc=False),
    )
    def k(tbl_hbm, idx_hbm, out_hbm, idx_v, out_v, sem):
        wid = lax.axis_index("s") * _NC + lax.axis_index("c")
        pltpu.sync_copy(idx_hbm.at[wid], idx_v)

        def grp(c, carry):
            copies = [
                pltpu.async_copy(
                    tbl_hbm.at[idx_v.at[c * _GRP + k_]],
                    out_v.at[c * _GRP + k_], sem)
                for k_ in range(_GRP)
            ]
            for cp in copies:
                cp.wait()
            return carry

        lax.fori_loop(0, _CPW // _GRP, grp, 0)
        pltpu.sync_copy(out_v, out_hbm.at[wid])

    return k(tbl1d, eidx3)


def _dot(a, b, dn):
    return lax.dot_general(a, b, (dn, ((), ())),
                           preferred_element_type=jnp.float32)


def _tc_body(dx, ea, eb, tgt, wb1, bb1, wb2, bb2, wb3, bb3, wb4, bb4,
             mx, mea, meb, wz2, wx, bt1, wt2, bt2, wt3, bt3, out):
    x = dx[...]
    x = jnp.maximum(_dot(x, wb1[...], ((1,), (1,))) + bb1[...], 0.0)
    x = jnp.maximum(_dot(x, wb2[...], ((1,), (1,))) + bb2[...], 0.0)
    x = jnp.maximum(_dot(x, wb3[...], ((1,), (1,))) + bb3[...], 0.0)
    x4 = jnp.maximum(_dot(x, wb4[...], ((1,), (1,))) + bb4[...], 0.0)

    tt = (_dot(mx[...], x4, ((1,), (1,)))
          + _dot(mea[...], ea[...], ((1,), (0,)))
          + _dot(meb[...], eb[...], ((1,), (0,))))
    a = tt.reshape(_D, _NEP, _BB)

    z = None
    for d in range(_D):
        ad = a[d]
        term = ad[:, None, :] * ad[None, :, :]
        z = term if z is None else z + term
    zt = z.reshape(_ZW, _BB)

    h = _dot(wx[...], x4, ((1,), (1,))) + _dot(wz2[...], zt, ((1,), (0,)))
    h = jnp.maximum(h + bt1[...], 0.0)
    h = jnp.maximum(_dot(wt2[...], h, ((1,), (0,))) + bt2[...], 0.0)
    logit = _dot(wt3[...], h, ((1,), (0,))) + bt3[...]
    p = jax.nn.sigmoid(logit)

    t = tgt[0]
    eps = 1e-12
    c = (t * jnp.log(jnp.clip(p, eps, 1.0))
         + (1.0 - t) * jnp.log(jnp.clip(1.0 - p, eps, 1.0)))
    s = jnp.sum(c, axis=1, keepdims=True) * (-1.0 / _B)

    @pl.when(pl.program_id(0) == 0)
    def _():
        out[...] = jnp.zeros_like(s)

    out[...] += s


def _tc_loss(dxB, embA, embB, tgt3, weights, interpret=False):
    full = lambda arr: pl.BlockSpec(arr.shape, lambda i: (0,) * arr.ndim)
    in_specs = [
        pl.BlockSpec((_BB, 13), lambda i: (i, 0)),
        pl.BlockSpec((_NTH * _D, _BB), lambda i: (0, i)),
        pl.BlockSpec((_NTH * _D, _BB), lambda i: (0, i)),
        pl.BlockSpec((1, 1, _BB), lambda i: (i, 0, 0)),
    ] + [full(w) for w in weights]
    return pl.pallas_call(
        _tc_body,
        grid=(_GRID,),
        in_specs=in_specs,
        out_specs=pl.BlockSpec((1, 1), lambda i: (0, 0)),
        out_shape=jax.ShapeDtypeStruct((1, 1), jnp.float32),
        interpret=interpret,
    )(dxB, embA, embB, tgt3, *weights)


def _selection_mats():
    mx = np.zeros((_D * _NEP, _D), np.float32)
    d = np.arange(_D)
    mx[d * _NEP, d] = 1.0
    me = np.zeros((_D * _NEP, _NT * _D), np.float32)
    ent, dd = np.meshgrid(np.arange(1, _NE), d, indexing="ij")
    me[dd.ravel() * _NEP + ent.ravel(), (ent.ravel() - 1) * _D + dd.ravel()] = 1.0
    return jnp.asarray(mx), jnp.asarray(me)


def kernel(dense_x, lS_i, target, emb_tables, bot_Ws, bot_bs, top_Ws, top_bs):
    doff = (jnp.arange(_D, dtype=jnp.int32) * _V)[None, :, None]
    toff = (jnp.arange(_NTH, dtype=jnp.int32) * (_D * _V))[:, None, None]
    embs = []
    for h in range(_NSPLIT):
        tbl1d = emb_tables[h * _NTH:(h + 1) * _NTH].transpose(0, 2, 1).reshape(-1)
        vv = lS_i[h * _NTH:(h + 1) * _NTH][:, None, :]
        eidx3 = (vv + toff + doff).reshape(_NW, _CPW, _CH)
        embs.append(_sc_gather(tbl1d, eidx3).reshape(_NTH * _D, _B))

    tgt3 = target.reshape(_GRID, 1, _BB)
    mx, me = _selection_mats()
    cols = np.array([i * _NEP + j for i in range(_NE) for j in range(i)])
    wz2 = jnp.zeros((512, _ZW), jnp.float32).at[:, cols].set(top_Ws[0][:, _D:])
    wx = top_Ws[0][:, :_D]
    weights = [
        bot_Ws[0], bot_bs[0][None, :], bot_Ws[1], bot_bs[1][None, :],
        bot_Ws[2], bot_bs[2][None, :], bot_Ws[3], bot_bs[3][None, :],
        mx, me[:, :_NTH * _D], me[:, _NTH * _D:], wz2, wx, top_bs[0][:, None],
        top_Ws[1], top_bs[1][:, None], top_Ws[2], top_bs[2][:, None],
    ]
    loss = _tc_loss(dense_x, embs[0], embs[1], tgt3, weights)
    return loss[0, 0]


# speedup vs baseline: 1.3379x; 1.1699x over previous
import functools

import numpy as np
import jax
import jax.numpy as jnp
from jax import lax
from jax.experimental import pallas as pl
from jax.experimental.pallas import tpu as pltpu
from jax.experimental.pallas import tpu_sc as plsc

_B = 4096
_NT = 26
_V = 100000
_D = 16
_NE = _NT + 1
_NEP = 32
_ZW = _NEP * _NEP
_BB = 512
_GRID = _B // _BB

_NC, _NS = 2, 16
_NW = _NC * _NS
_CH = 128
_EPS = _B * _NT * _D
_CPW = _EPS // _NW // _CH
_GRP = 16


def _sc_gather(tbl1d, eidx3):
    mesh = plsc.VectorSubcoreMesh(core_axis_name="c", subcore_axis_name="s")

    @functools.partial(
        pl.kernel,
        mesh=mesh,
        out_type=jax.ShapeDtypeStruct((_NW, _CPW, _CH), jnp.float32),
        scratch_types=[
            pltpu.VMEM((_CPW, _CH), jnp.int32),
            pltpu.VMEM((_CPW, _CH), jnp.float32),
            pltpu.SemaphoreType.DMA,
        ],
        compiler_params=pltpu.CompilerParams(use_tc_tiling_on_sc=False),
    )
    def k(tbl_hbm, idx_hbm, out_hbm, idx_v, out_v, sem):
        wid = lax.axis_index("s") * _NC + lax.axis_index("c")
        pltpu.sync_copy(idx_hbm.at[wid], idx_v)

        def grp(c, carry):
            copies = [
                pltpu.async_copy(
                    tbl_hbm.at[idx_v.at[c * _GRP + k_]],
                    out_v.at[c * _GRP + k_], sem)
                for k_ in range(_GRP)
            ]
            for cp in copies:
                cp.wait()
            return carry

        lax.fori_loop(0, _CPW // _GRP, grp, 0)
        pltpu.sync_copy(out_v, out_hbm.at[wid])

    return k(tbl1d, eidx3)


def _dot(a, b, dn):
    return lax.dot_general(a, b, (dn, ((), ())),
                           preferred_element_type=jnp.float32)


def _tc_body(dx, e, tgt, wb1, bb1, wb2, bb2, wb3, bb3, wb4, bb4,
             mx, me, wz2, wx, bt1, wt2, bt2, wt3, bt3, out):
    x = dx[...]
    x = jnp.maximum(_dot(x, wb1[...], ((1,), (1,))) + bb1[...], 0.0)
    x = jnp.maximum(_dot(x, wb2[...], ((1,), (1,))) + bb2[...], 0.0)
    x = jnp.maximum(_dot(x, wb3[...], ((1,), (1,))) + bb3[...], 0.0)
    x4 = jnp.maximum(_dot(x, wb4[...], ((1,), (1,))) + bb4[...], 0.0)

    tt = (_dot(mx[...], x4, ((1,), (1,)))
          + _dot(me[...], e[...], ((1,), (0,))))
    a = tt.reshape(_D, _NEP, _BB)

    z = None
    for d in range(_D):
        ad = a[d]
        term = ad[:, None, :] * ad[None, :, :]
        z = term if z is None else z + term
    zt = z.reshape(_ZW, _BB)

    h = _dot(wx[...], x4, ((1,), (1,))) + _dot(wz2[...], zt, ((1,), (0,)))
    h = jnp.maximum(h + bt1[...], 0.0)
    h = jnp.maximum(_dot(wt2[...], h, ((1,), (0,))) + bt2[...], 0.0)
    logit = _dot(wt3[...], h, ((1,), (0,))) + bt3[...]
    p = jax.nn.sigmoid(logit)

    t = tgt[0]
    eps = 1e-12
    c = (t * jnp.log(jnp.clip(p, eps, 1.0))
         + (1.0 - t) * jnp.log(jnp.clip(1.0 - p, eps, 1.0)))
    s = jnp.sum(c, axis=1, keepdims=True) * (-1.0 / _B)

    @pl.when(pl.program_id(0) == 0)
    def _():
        out[...] = jnp.zeros_like(s)

    out[...] += s


def _tc_loss(dxB, emb, tgt3, weights, interpret=False):
    full = lambda arr: pl.BlockSpec(arr.shape, lambda i: (0,) * arr.ndim)
    in_specs = [
        pl.BlockSpec((_BB, 13), lambda i: (i, 0)),
        pl.BlockSpec((_NT * _D, _BB), lambda i: (0, i)),
        pl.BlockSpec((1, 1, _BB), lambda i: (i, 0, 0)),
    ] + [full(w) for w in weights]
    return pl.pallas_call(
        _tc_body,
        grid=(_GRID,),
        in_specs=in_specs,
        out_specs=pl.BlockSpec((1, 1), lambda i: (0, 0)),
        out_shape=jax.ShapeDtypeStruct((1, 1), jnp.float32),
        interpret=interpret,
    )(dxB, emb, tgt3, *weights)


def _selection_mats():
    mx = np.zeros((_D * _NEP, _D), np.float32)
    d = np.arange(_D)
    mx[d * _NEP, d] = 1.0
    me = np.zeros((_D * _NEP, _NT * _D), np.float32)
    ent, dd = np.meshgrid(np.arange(1, _NE), d, indexing="ij")
    me[dd.ravel() * _NEP + ent.ravel(), (ent.ravel() - 1) * _D + dd.ravel()] = 1.0
    return jnp.asarray(mx), jnp.asarray(me)


def kernel(dense_x, lS_i, target, emb_tables, bot_Ws, bot_bs, top_Ws, top_bs):
    tbl1d = emb_tables.transpose(0, 2, 1).reshape(-1)
    doff = (jnp.arange(_D, dtype=jnp.int32) * _V)[None, :, None]
    toff = (jnp.arange(_NT, dtype=jnp.int32) * (_D * _V))[:, None, None]
    eidx3 = (lS_i[:, None, :] + toff + doff).reshape(_NW, _CPW, _CH)
    emb = _sc_gather(tbl1d, eidx3).reshape(_NT * _D, _B)

    tgt3 = target.reshape(_GRID, 1, _BB)
    mx, me = _selection_mats()
    cols = np.array([i * _NEP + j for i in range(_NE) for j in range(i)])
    wz2 = jnp.zeros((512, _ZW), jnp.float32).at[:, cols].set(top_Ws[0][:, _D:])
    wx = top_Ws[0][:, :_D]
    weights = [
        bot_Ws[0], bot_bs[0][None, :], bot_Ws[1], bot_bs[1][None, :],
        bot_Ws[2], bot_bs[2][None, :], bot_Ws[3], bot_bs[3][None, :],
        mx, me, wz2, wx, top_bs[0][:, None],
        top_Ws[1], top_bs[1][:, None], top_Ws[2], top_bs[2][:, None],
    ]
    loss = _tc_loss(dense_x, emb, tgt3, weights)
    return loss[0, 0]


# speedup vs baseline: 1.3514x; 1.0101x over previous
import functools

import numpy as np
import jax
import jax.numpy as jnp
from jax import lax
from jax.experimental import pallas as pl
from jax.experimental.pallas import tpu as pltpu
from jax.experimental.pallas import tpu_sc as plsc

_B = 4096
_NT = 26
_V = 100000
_D = 16
_NE = _NT + 1
_NEP = 32
_ZW = _NEP * _NEP
_BB = 512
_GRID = _B // _BB

_NC, _NS = 2, 16
_NW = _NC * _NS
_CH = 128
_EPS = _B * _NT * _D
_CPW = _EPS // _NW // _CH
_GRP = 16


def _sc_gather(tbl1d, eidx3):
    mesh = plsc.VectorSubcoreMesh(core_axis_name="c", subcore_axis_name="s")

    @functools.partial(
        pl.kernel,
        mesh=mesh,
        out_type=jax.ShapeDtypeStruct((_NW, _CPW, _CH), jnp.float32),
        scratch_types=[
            pltpu.VMEM((_CPW, _CH), jnp.int32),
            pltpu.VMEM((_CPW, _CH), jnp.float32),
            pltpu.SemaphoreType.DMA,
        ],
        compiler_params=pltpu.CompilerParams(use_tc_tiling_on_sc=False),
    )
    def k(tbl_hbm, idx_hbm, out_hbm, idx_v, out_v, sem):
        wid = lax.axis_index("s") * _NC + lax.axis_index("c")
        pltpu.sync_copy(idx_hbm.at[wid], idx_v)

        def grp(c, carry):
            for k_ in range(_GRP):
                pltpu.async_copy(
                    tbl_hbm.at[idx_v.at[c * _GRP + k_]],
                    out_v.at[c * _GRP + k_], sem)

            @pl.when(c > 0)
            def _():
                for k_ in range(_GRP):
                    pltpu.make_async_copy(
                        tbl_hbm.at[idx_v.at[(c - 1) * _GRP + k_]],
                        out_v.at[(c - 1) * _GRP + k_], sem).wait()

            return carry

        ngrp = _CPW // _GRP
        lax.fori_loop(0, ngrp, grp, 0)
        for k_ in range(_GRP):
            pltpu.make_async_copy(
                tbl_hbm.at[idx_v.at[(ngrp - 1) * _GRP + k_]],
                out_v.at[(ngrp - 1) * _GRP + k_], sem).wait()
        pltpu.sync_copy(out_v, out_hbm.at[wid])

    return k(tbl1d, eidx3)


def _dot(a, b, dn):
    return lax.dot_general(a, b, (dn, ((), ())),
                           preferred_element_type=jnp.float32)


def _tc_body(dx, e, tgt, wb1, bb1, wb2, bb2, wb3, bb3, wb4, bb4,
             mx, me, wz2, wx, bt1, wt2, bt2, wt3, bt3, out):
    x = dx[...]
    x = jnp.maximum(_dot(x, wb1[...], ((1,), (1,))) + bb1[...], 0.0)
    x = jnp.maximum(_dot(x, wb2[...], ((1,), (1,))) + bb2[...], 0.0)
    x = jnp.maximum(_dot(x, wb3[...], ((1,), (1,))) + bb3[...], 0.0)
    x4 = jnp.maximum(_dot(x, wb4[...], ((1,), (1,))) + bb4[...], 0.0)

    tt = (_dot(mx[...], x4, ((1,), (1,)))
          + _dot(me[...], e[...], ((1,), (0,))))
    a = tt.reshape(_D, _NEP, _BB)

    z = None
    for d in range(_D):
        ad = a[d]
        term = ad[:, None, :] * ad[None, :, :]
        z = term if z is None else z + term
    zt = z.reshape(_ZW, _BB)

    h = _dot(wx[...], x4, ((1,), (1,))) + _dot(wz2[...], zt, ((1,), (0,)))
    h = jnp.maximum(h + bt1[...], 0.0)
    h = jnp.maximum(_dot(wt2[...], h, ((1,), (0,))) + bt2[...], 0.0)
    logit = _dot(wt3[...], h, ((1,), (0,))) + bt3[...]
    p = jax.nn.sigmoid(logit)

    t = tgt[0]
    eps = 1e-12
    c = (t * jnp.log(jnp.clip(p, eps, 1.0))
         + (1.0 - t) * jnp.log(jnp.clip(1.0 - p, eps, 1.0)))
    s = jnp.sum(c, axis=1, keepdims=True) * (-1.0 / _B)

    @pl.when(pl.program_id(0) == 0)
    def _():
        out[...] = jnp.zeros_like(s)

    out[...] += s


def _tc_loss(dxB, emb, tgt3, weights, interpret=False):
    full = lambda arr: pl.BlockSpec(arr.shape, lambda i: (0,) * arr.ndim)
    in_specs = [
        pl.BlockSpec((_BB, 13), lambda i: (i, 0)),
        pl.BlockSpec((_NT * _D, _BB), lambda i: (0, i)),
        pl.BlockSpec((1, 1, _BB), lambda i: (i, 0, 0)),
    ] + [full(w) for w in weights]
    return pl.pallas_call(
        _tc_body,
        grid=(_GRID,),
        in_specs=in_specs,
        out_specs=pl.BlockSpec((1, 1), lambda i: (0, 0)),
        out_shape=jax.ShapeDtypeStruct((1, 1), jnp.float32),
        interpret=interpret,
    )(dxB, emb, tgt3, *weights)


def _selection_mats():
    mx = np.zeros((_D * _NEP, _D), np.float32)
    d = np.arange(_D)
    mx[d * _NEP, d] = 1.0
    me = np.zeros((_D * _NEP, _NT * _D), np.float32)
    ent, dd = np.meshgrid(np.arange(1, _NE), d, indexing="ij")
    me[dd.ravel() * _NEP + ent.ravel(), (ent.ravel() - 1) * _D + dd.ravel()] = 1.0
    return jnp.asarray(mx), jnp.asarray(me)


def kernel(dense_x, lS_i, target, emb_tables, bot_Ws, bot_bs, top_Ws, top_bs):
    tbl1d = emb_tables.transpose(0, 2, 1).reshape(-1)
    doff = (jnp.arange(_D, dtype=jnp.int32) * _V)[None, :, None]
    toff = (jnp.arange(_NT, dtype=jnp.int32) * (_D * _V))[:, None, None]
    eidx3 = (lS_i[:, None, :] + toff + doff).reshape(_NW, _CPW, _CH)
    emb = _sc_gather(tbl1d, eidx3).reshape(_NT * _D, _B)

    tgt3 = target.reshape(_GRID, 1, _BB)
    mx, me = _selection_mats()
    cols = np.array([i * _NEP + j for i in range(_NE) for j in range(i)])
    wz2 = jnp.zeros((512, _ZW), jnp.float32).at[:, cols].set(top_Ws[0][:, _D:])
    wx = top_Ws[0][:, :_D]
    weights = [
        bot_Ws[0], bot_bs[0][None, :], bot_Ws[1], bot_bs[1][None, :],
        bot_Ws[2], bot_bs[2][None, :], bot_Ws[3], bot_bs[3][None, :],
        mx, me, wz2, wx, top_bs[0][:, None],
        top_Ws[1], top_bs[1][:, None], top_Ws[2], top_bs[2][:, None],
    ]
    loss = _tc_loss(dense_x, emb, tgt3, weights)
    return loss[0, 0]
